# baseline (speedup 1.0000x reference)
import dataclasses
import functools

import jax
import jax.numpy as jnp
from jax import lax
from jax.experimental import pallas as pl
from jax.experimental.pallas import tpu as pltpu
from jax.experimental.pallas import tpu_sc as plsc

N = 10000
NP = 10240
E = 320000
NTILES = 32
CH = 80
NCH = 128
EPT = NCH * CH
RPT = NP // 16

D = 128
D_IN = 128
D_HID = 64
D_OUT = 121

_SC_PARAMS = pltpu.CompilerParams()
if "needs_layout_passes" in pltpu.CompilerParams.__dataclass_fields__:
  _SC_PARAMS = dataclasses.replace(_SC_PARAMS, needs_layout_passes=False)


def _make_sc_agg(DG, DS, den_col, sc_params):
  dblk, dlane = den_col // 16, den_col % 16
  inplace = DG == DS
  mesh = plsc.VectorSubcoreMesh(core_axis_name="c", subcore_axis_name="s")

  @functools.partial(
      pl.kernel,
      compiler_params=sc_params,
      out_type=jax.ShapeDtypeStruct((2, NP, DS), jnp.float32),
      mesh=mesh,
      scratch_types=[
          pltpu.VMEM((NP,), jnp.float32),
          pltpu.VMEM((NP,), jnp.float32),
          pltpu.VMEM((2, CH), jnp.int32),
          pltpu.VMEM((2, CH), jnp.int32),
          pltpu.VMEM((1, CH), jnp.int32),
          pltpu.VMEM((1, CH), jnp.int32),
          pltpu.VMEM((CH,), jnp.float32),
          pltpu.VMEM((CH,), jnp.float32),
          pltpu.VMEM((CH, DG), jnp.float32),
          pltpu.VMEM((CH, DG), jnp.float32),
          (pltpu.VMEM((CH, DS), jnp.float32)
           if not inplace else pltpu.VMEM((1, 16), jnp.float32)),
          (pltpu.VMEM((CH, DS), jnp.float32)
           if not inplace else pltpu.VMEM((1, 16), jnp.float32)),
          pltpu.VMEM_SHARED((NP, DS), jnp.float32),
          pltpu.SemaphoreType.DMA,
          pltpu.SemaphoreType.DMA,
          pltpu.SemaphoreType.DMA,
          pltpu.SemaphoreType.DMA,
          pltpu.SemaphoreType.DMA,
          pltpu.SemaphoreType.DMA,
      ],
  )
  def sc_agg(edg_hbm, asv_hbm, adv_hbm, h_hbm, out_hbm,
             as_v, ad_v, ebA, ebB, dbA, dbB, exbA, exbB, gbA, gbB,
             sbA_, sbB_, acc,
             isemA, isemB, gsemA, gsemB, ssemA, ssemB):
    sbA = gbA if inplace else sbA_
    sbB = gbB if inplace else sbB_
    cid = lax.axis_index("c")
    sid = lax.axis_index("s")
    wid = cid * 16 + sid

    pltpu.sync_copy(asv_hbm, as_v)
    pltpu.sync_copy(adv_hbm, ad_v)

    zero16 = jnp.zeros((16,), jnp.float32)
    dmask = jnp.where(lax.iota(jnp.int32, 16) == dlane, 1.0, 0.0)

    @pl.loop(0, CH)
    def _zrow(j):
      for t in range(DS // 16):
        sbA[j, pl.ds(t * 16, 16)] = zero16

    for m in range(RPT // CH):
      pltpu.sync_copy(sbA, acc.at[pl.ds(sid * RPT + m * CH, CH)])
    plsc.subcore_barrier()

    def do_chunk(c, eb, db, exb, gb, sb, isem, gsem, ssem,
                 ebn, dbn, gbn, sbn, isemn, gsemn, ssemn):
      for k in range(0, CH, 16):
        si = eb[0, pl.ds(k, 16)]
        di = eb[1, pl.ds(k, 16)]
        db[0, pl.ds(k, 16)] = di
        e = plsc.load_gather(as_v, [si]) + plsc.load_gather(ad_v, [di])
        e = jnp.where(e >= 0.0, e, 0.2 * e)
        exb[pl.ds(k, 16)] = jnp.exp(e)

      pltpu.make_async_copy(h_hbm.at[eb.at[0]], gb, gsem).wait()

      @pl.when(c + 2 < NCH)
      def _():
        pltpu.async_copy(edg_hbm.at[wid, c + 2], eb, isem)

      @pl.when(c + 1 < NCH)
      def _():
        pltpu.make_async_copy(edg_hbm.at[wid, c + 1], ebn, isemn).wait()

        @pl.when(c >= 1)
        def _():
          pltpu.make_async_copy(sbn, acc.at[dbn.at[0]], ssemn).wait()

        pltpu.async_copy(h_hbm.at[ebn.at[0]], gbn, gsemn)

      @pl.loop(0, CH, step=2)
      def _scale(j):
        ex0 = plsc.load_gather(exb, [jnp.zeros((16,), jnp.int32) + j])
        ex1 = plsc.load_gather(exb, [jnp.zeros((16,), jnp.int32) + (j + 1)])
        for t in range(dblk + 1):
          if t * 16 < DG:
            p0 = gb[j, pl.ds(t * 16, 16)] * ex0
            p1 = gb[j + 1, pl.ds(t * 16, 16)] * ex1
          else:
            p0 = zero16
            p1 = zero16
          if t == dblk:
            p0 = p0 + ex0 * dmask
            p1 = p1 + ex1 * dmask
          sb[j, pl.ds(t * 16, 16)] = p0
          sb[j + 1, pl.ds(t * 16, 16)] = p1

      pltpu.async_copy(sb, acc.at[db.at[0]], ssem, add=True)

    pltpu.async_copy(edg_hbm.at[wid, 0], ebA, isemA)
    pltpu.async_copy(edg_hbm.at[wid, 1], ebB, isemB)
    pltpu.make_async_copy(edg_hbm.at[wid, 0], ebA, isemA).wait()
    pltpu.async_copy(h_hbm.at[ebA.at[0]], gbA, gsemA)

    @pl.loop(0, NCH, step=2)
    def _chunks(c):
      do_chunk(c, ebA, dbA, exbA, gbA, sbA, isemA, gsemA, ssemA,
               ebB, dbB, gbB, sbB, isemB, gsemB, ssemB)
      do_chunk(c + 1, ebB, dbB, exbB, gbB, sbB, isemB, gsemB, ssemB,
               ebA, dbA, gbA, sbA, isemA, gsemA, ssemA)

    pltpu.make_async_copy(sbA, acc.at[dbA.at[0]], ssemA).wait()
    pltpu.make_async_copy(sbB, acc.at[dbB.at[0]], ssemB).wait()

    plsc.subcore_barrier()
    pltpu.sync_copy(acc.at[pl.ds(sid * RPT, RPT)],
                    out_hbm.at[cid, pl.ds(sid * RPT, RPT)])

  return sc_agg


_SC_PARAMS_LINEAR = dataclasses.replace(_SC_PARAMS, use_tc_tiling_on_sc=False)

_sc_agg1 = _make_sc_agg(D_HID, D_HID + 16, D_HID, _SC_PARAMS_LINEAR)
_sc_agg2 = _make_sc_agg(D, D, D_OUT, _SC_PARAMS)


def _tc_front_body(x_ref, w_ref, asr, adr, h_ref, asc, adc):
  h = jnp.dot(x_ref[...], w_ref[...], preferred_element_type=jnp.float32,
              precision=lax.Precision.HIGHEST)
  h_ref[...] = h
  asc[...] = jnp.sum(h * asr[...], axis=1, keepdims=True)
  adc[...] = jnp.sum(h * adr[...], axis=1, keepdims=True)


def _tc_front(x_pad, W1p, a1s, a1d):
  R = 1024
  return pl.pallas_call(
      _tc_front_body,
      grid=(NP // R,),
      in_specs=[
          pl.BlockSpec((R, D_IN), lambda i: (i, 0)),
          pl.BlockSpec((D_IN, D_HID), lambda i: (0, 0)),
          pl.BlockSpec((1, D_HID), lambda i: (0, 0)),
          pl.BlockSpec((1, D_HID), lambda i: (0, 0)),
      ],
      out_specs=[
          pl.BlockSpec((R, D_HID), lambda i: (i, 0)),
          pl.BlockSpec((R, 1), lambda i: (i, 0)),
          pl.BlockSpec((R, 1), lambda i: (i, 0)),
      ],
      out_shape=[
          jax.ShapeDtypeStruct((NP, D_HID), jnp.float32),
          jax.ShapeDtypeStruct((NP, 1), jnp.float32),
          jax.ShapeDtypeStruct((NP, 1), jnp.float32),
      ],
  )(x_pad, W1p, a1s, a1d)


def _tc_mid_body(acca, accb, asc, adc, h1, b1r, w2, a2s, a2d,
                 h2_ref, as2, ad2):
  e = asc[...] + adc[...]
  base = jnp.exp(jnp.where(e >= 0.0, e, 0.2 * e))
  num = acca[:, :D_HID] + accb[:, :D_HID] + base * h1[...]
  den = (acca[:, D_HID:D_HID + 1] + accb[:, D_HID:D_HID + 1] + base + 1e-16)
  g = jnp.maximum(num / den + b1r[...], 0.0)
  h2 = jnp.dot(g, w2[...], preferred_element_type=jnp.float32,
               precision=lax.Precision.HIGHEST)
  h2_ref[...] = h2
  as2[...] = jnp.sum(h2 * a2s[...], axis=1, keepdims=True)
  ad2[...] = jnp.sum(h2 * a2d[...], axis=1, keepdims=True)


def _tc_mid(acc_a, acc_b, as1, ad1, h1, b1r, W2p, a2s, a2d):
  R = 1024
  DS1 = D_HID + 16
  return pl.pallas_call(
      _tc_mid_body,
      grid=(NP // R,),
      in_specs=[
          pl.BlockSpec((R, DS1), lambda i: (i, 0)),
          pl.BlockSpec((R, DS1), lambda i: (i, 0)),
          pl.BlockSpec((R, 1), lambda i: (i, 0)),
          pl.BlockSpec((R, 1), lambda i: (i, 0)),
          pl.BlockSpec((R, D_HID), lambda i: (i, 0)),
          pl.BlockSpec((1, D_HID), lambda i: (0, 0)),
          pl.BlockSpec((D_HID, D), lambda i: (0, 0)),
          pl.BlockSpec((1, D), lambda i: (0, 0)),
          pl.BlockSpec((1, D), lambda i: (0, 0)),
      ],
      out_specs=[
          pl.BlockSpec((R, D), lambda i: (i, 0)),
          pl.BlockSpec((R, 1), lambda i: (i, 0)),
          pl.BlockSpec((R, 1), lambda i: (i, 0)),
      ],
      out_shape=[
          jax.ShapeDtypeStruct((NP, D), jnp.float32),
          jax.ShapeDtypeStruct((NP, 1), jnp.float32),
          jax.ShapeDtypeStruct((NP, 1), jnp.float32),
      ],
  )(acc_a, acc_b, as1, ad1, h1, b1r, W2p, a2s, a2d)


def _tc_final_body(acca, accb, asc, adc, h2, b2r, out_ref):
  e = asc[...] + adc[...]
  base = jnp.exp(jnp.where(e >= 0.0, e, 0.2 * e))
  num = acca[...] + accb[...] + base * h2[...]
  den = (acca[:, D_OUT:D_OUT + 1] + accb[:, D_OUT:D_OUT + 1] + base + 1e-16)
  z = num / den + b2r[...]
  out_ref[...] = 1.0 / (1.0 + jnp.exp(-z))


def _tc_final(acc_a, acc_b, as2, ad2, h2, b2r):
  R = 1024
  return pl.pallas_call(
      _tc_final_body,
      grid=(NP // R,),
      in_specs=[
          pl.BlockSpec((R, D), lambda i: (i, 0)),
          pl.BlockSpec((R, D), lambda i: (i, 0)),
          pl.BlockSpec((R, 1), lambda i: (i, 0)),
          pl.BlockSpec((R, 1), lambda i: (i, 0)),
          pl.BlockSpec((R, D), lambda i: (i, 0)),
          pl.BlockSpec((1, D), lambda i: (0, 0)),
      ],
      out_specs=pl.BlockSpec((R, D), lambda i: (i, 0)),
      out_shape=jax.ShapeDtypeStruct((NP, D), jnp.float32),
  )(acc_a, acc_b, as2, ad2, h2, b2r)


def kernel(x, edge_index, W1, a_src1, a_dst1, b1, W2, a_src2, a_dst2, b2):
  x_pad = jnp.zeros((NP, D_IN), jnp.float32).at[:N].set(x)
  npad = NTILES * EPT - E
  pad_idx = N + (jnp.arange(npad, dtype=jnp.int32) % (NP - N))
  srcp = jnp.concatenate([edge_index[0].astype(jnp.int32), pad_idx])
  dstp = jnp.concatenate([edge_index[1].astype(jnp.int32), pad_idx])
  edg = jnp.stack([srcp.reshape(NTILES, NCH, CH),
                   dstp.reshape(NTILES, NCH, CH)], axis=2)

  W1p = W1
  a1sp = a_src1.reshape(1, D_HID)
  a1dp = a_dst1.reshape(1, D_HID)
  b1r = b1.reshape(1, D_HID)
  W2p = jnp.zeros((D_HID, D), jnp.float32).at[:, :D_OUT].set(W2)
  a2sp = jnp.zeros((1, D), jnp.float32).at[0, :D_OUT].set(a_src2)
  a2dp = jnp.zeros((1, D), jnp.float32).at[0, :D_OUT].set(a_dst2)
  b2r = jnp.zeros((1, D), jnp.float32).at[0, :D_OUT].set(b2)

  h1, as1, ad1 = _tc_front(x_pad, W1p, a1sp, a1dp)
  acc1 = _sc_agg1(edg, as1.reshape(NP), ad1.reshape(NP), h1)
  h2, as2, ad2 = _tc_mid(acc1[0], acc1[1], as1, ad1, h1, b1r, W2p,
                         a2sp, a2dp)
  acc2 = _sc_agg2(edg, as2.reshape(NP), ad2.reshape(NP), h2)
  outp = _tc_final(acc2[0], acc2[1], as2, ad2, h2, b2r)
  return outp[:N, :D_OUT]


# speedup vs baseline: 1.0954x; 1.0954x over previous
import dataclasses
import functools

import jax
import jax.numpy as jnp
from jax import lax
from jax.experimental import pallas as pl
from jax.experimental.pallas import tpu as pltpu
from jax.experimental.pallas import tpu_sc as plsc

N = 10000
NP = 10112
E = 320000
NTILES = 32
CH = 96
NCH = 106
EPT = NCH * CH
RPT = NP // 16

D = 128
D_IN = 128
D_HID = 64
D_OUT = 121

_SC_PARAMS = pltpu.CompilerParams()
if "needs_layout_passes" in pltpu.CompilerParams.__dataclass_fields__:
  _SC_PARAMS = dataclasses.replace(_SC_PARAMS, needs_layout_passes=False)


def _make_sc_agg(den_col):
  dblk, dlane = den_col // 16, den_col % 16
  mesh = plsc.VectorSubcoreMesh(core_axis_name="c", subcore_axis_name="s")

  @functools.partial(
      pl.kernel,
      compiler_params=_SC_PARAMS,
      out_type=jax.ShapeDtypeStruct((2, NP, D), jnp.float32),
      mesh=mesh,
      scratch_types=[
          pltpu.VMEM((NP,), jnp.float32),
          pltpu.VMEM((NP,), jnp.float32),
          pltpu.VMEM((2, CH), jnp.int32),
          pltpu.VMEM((2, CH), jnp.int32),
          pltpu.VMEM((1, CH), jnp.int32),
          pltpu.VMEM((1, CH), jnp.int32),
          pltpu.VMEM((CH,), jnp.float32),
          pltpu.VMEM((CH,), jnp.float32),
          pltpu.VMEM((CH, D), jnp.float32),
          pltpu.VMEM((CH, D), jnp.float32),
          pltpu.VMEM_SHARED((NP, D), jnp.float32),
          pltpu.SemaphoreType.DMA,
          pltpu.SemaphoreType.DMA,
          pltpu.SemaphoreType.DMA,
          pltpu.SemaphoreType.DMA,
          pltpu.SemaphoreType.DMA,
          pltpu.SemaphoreType.DMA,
      ],
  )
  def sc_agg(edg_hbm, asv_hbm, adv_hbm, h_hbm, out_hbm,
             as_v, ad_v, ebA, ebB, dbA, dbB, exbA, exbB, gbA, gbB, acc,
             isemA, isemB, gsemA, gsemB, ssemA, ssemB):
    cid = lax.axis_index("c")
    sid = lax.axis_index("s")
    wid = cid * 16 + sid

    pltpu.sync_copy(asv_hbm, as_v)
    pltpu.sync_copy(adv_hbm, ad_v)

    zero16 = jnp.zeros((16,), jnp.float32)
    dmask = jnp.where(lax.iota(jnp.int32, 16) == dlane, 1.0, 0.0)

    @pl.loop(0, CH)
    def _zrow(j):
      for t in range(D // 16):
        gbA[j, pl.ds(t * 16, 16)] = zero16

    for m in range(RPT // CH):
      pltpu.sync_copy(gbA, acc.at[pl.ds(sid * RPT + m * CH, CH)])
    rem = RPT - (RPT // CH) * CH
    if rem:
      pltpu.sync_copy(gbA.at[pl.ds(0, rem)],
                      acc.at[pl.ds(sid * RPT + (RPT // CH) * CH, rem)])
    plsc.subcore_barrier()

    def do_chunk(c, eb, db, exb, gb, isem, gsem, ssem,
                 ebn, dbn, gbn, isemn, gsemn, ssemn):
      for k in range(0, CH, 16):
        si = eb[0, pl.ds(k, 16)]
        di = eb[1, pl.ds(k, 16)]
        db[0, pl.ds(k, 16)] = di
        e = plsc.load_gather(as_v, [si]) + plsc.load_gather(ad_v, [di])
        e = jnp.where(e >= 0.0, e, 0.2 * e)
        exb[pl.ds(k, 16)] = jnp.exp(e)

      pltpu.make_async_copy(h_hbm.at[eb.at[0]], gb, gsem).wait()

      @pl.when(c + 2 < NCH)
      def _():
        pltpu.async_copy(edg_hbm.at[wid, c + 2], eb, isem)

      @pl.when(c + 1 < NCH)
      def _():
        pltpu.make_async_copy(edg_hbm.at[wid, c + 1], ebn, isemn).wait()

        @pl.when(c >= 1)
        def _():
          pltpu.make_async_copy(gbn, acc.at[dbn.at[0]], ssemn).wait()

        pltpu.async_copy(h_hbm.at[ebn.at[0]], gbn, gsemn)

      @pl.loop(0, CH, step=2)
      def _scale(j):
        ex0 = plsc.load_gather(exb, [jnp.zeros((16,), jnp.int32) + j])
        ex1 = plsc.load_gather(exb, [jnp.zeros((16,), jnp.int32) + (j + 1)])
        for t in range(dblk + 1):
          p0 = gb[j, pl.ds(t * 16, 16)] * ex0
          p1 = gb[j + 1, pl.ds(t * 16, 16)] * ex1
          if t == dblk:
            p0 = p0 + ex0 * dmask
            p1 = p1 + ex1 * dmask
          gb[j, pl.ds(t * 16, 16)] = p0
          gb[j + 1, pl.ds(t * 16, 16)] = p1

      pltpu.async_copy(gb, acc.at[db.at[0]], ssem, add=True)

    pltpu.async_copy(edg_hbm.at[wid, 0], ebA, isemA)
    pltpu.async_copy(edg_hbm.at[wid, 1], ebB, isemB)
    pltpu.make_async_copy(edg_hbm.at[wid, 0], ebA, isemA).wait()
    pltpu.async_copy(h_hbm.at[ebA.at[0]], gbA, gsemA)

    @pl.loop(0, NCH, step=2)
    def _chunks(c):
      do_chunk(c, ebA, dbA, exbA, gbA, isemA, gsemA, ssemA,
               ebB, dbB, gbB, isemB, gsemB, ssemB)
      do_chunk(c + 1, ebB, dbB, exbB, gbB, isemB, gsemB, ssemB,
               ebA, dbA, gbA, isemA, gsemA, ssemA)

    pltpu.make_async_copy(gbA, acc.at[dbA.at[0]], ssemA).wait()
    pltpu.make_async_copy(gbB, acc.at[dbB.at[0]], ssemB).wait()

    plsc.subcore_barrier()
    pltpu.sync_copy(acc.at[pl.ds(sid * RPT, RPT)],
                    out_hbm.at[cid, pl.ds(sid * RPT, RPT)])

  return sc_agg


_sc_agg1 = _make_sc_agg(D_HID)
_sc_agg2 = _make_sc_agg(D_OUT)


def _tc_front_body(x_ref, w_ref, asr, adr, h_ref, asc, adc):
  h = jnp.dot(x_ref[...], w_ref[...], preferred_element_type=jnp.float32,
              precision=lax.Precision.HIGHEST)
  h_ref[...] = h
  asc[...] = jnp.sum(h * asr[...], axis=1, keepdims=True)
  adc[...] = jnp.sum(h * adr[...], axis=1, keepdims=True)


def _tc_front(x_pad, W1p, a1s, a1d):
  R = 1024
  return pl.pallas_call(
      _tc_front_body,
      grid=(NP // R,),
      in_specs=[
          pl.BlockSpec((R, D_IN), lambda i: (i, 0)),
          pl.BlockSpec((D_IN, D), lambda i: (0, 0)),
          pl.BlockSpec((1, D), lambda i: (0, 0)),
          pl.BlockSpec((1, D), lambda i: (0, 0)),
      ],
      out_specs=[
          pl.BlockSpec((R, D), lambda i: (i, 0)),
          pl.BlockSpec((R, 1), lambda i: (i, 0)),
          pl.BlockSpec((R, 1), lambda i: (i, 0)),
      ],
      out_shape=[
          jax.ShapeDtypeStruct((NP, D), jnp.float32),
          jax.ShapeDtypeStruct((NP, 1), jnp.float32),
          jax.ShapeDtypeStruct((NP, 1), jnp.float32),
      ],
  )(x_pad, W1p, a1s, a1d)


def _tc_mid_body(acca, accb, asc, adc, h1, b1r, w2, a2s, a2d,
                 h2_ref, as2, ad2):
  e = asc[...] + adc[...]
  base = jnp.exp(jnp.where(e >= 0.0, e, 0.2 * e))
  num = acca[:, :D_HID] + accb[:, :D_HID] + base * h1[:, :D_HID]
  den = (acca[:, D_HID:D_HID + 1] + accb[:, D_HID:D_HID + 1] + base + 1e-16)
  g = jnp.maximum(num / den + b1r[...], 0.0)
  h2 = jnp.dot(g, w2[...], preferred_element_type=jnp.float32,
               precision=lax.Precision.HIGHEST)
  h2_ref[...] = h2
  as2[...] = jnp.sum(h2 * a2s[...], axis=1, keepdims=True)
  ad2[...] = jnp.sum(h2 * a2d[...], axis=1, keepdims=True)


def _tc_mid(acc_a, acc_b, as1, ad1, h1, b1r, W2p, a2s, a2d):
  R = 1024
  return pl.pallas_call(
      _tc_mid_body,
      grid=(NP // R,),
      in_specs=[
          pl.BlockSpec((R, D), lambda i: (i, 0)),
          pl.BlockSpec((R, D), lambda i: (i, 0)),
          pl.BlockSpec((R, 1), lambda i: (i, 0)),
          pl.BlockSpec((R, 1), lambda i: (i, 0)),
          pl.BlockSpec((R, D), lambda i: (i, 0)),
          pl.BlockSpec((1, D_HID), lambda i: (0, 0)),
          pl.BlockSpec((D_HID, D), lambda i: (0, 0)),
          pl.BlockSpec((1, D), lambda i: (0, 0)),
          pl.BlockSpec((1, D), lambda i: (0, 0)),
      ],
      out_specs=[
          pl.BlockSpec((R, D), lambda i: (i, 0)),
          pl.BlockSpec((R, 1), lambda i: (i, 0)),
          pl.BlockSpec((R, 1), lambda i: (i, 0)),
      ],
      out_shape=[
          jax.ShapeDtypeStruct((NP, D), jnp.float32),
          jax.ShapeDtypeStruct((NP, 1), jnp.float32),
          jax.ShapeDtypeStruct((NP, 1), jnp.float32),
      ],
  )(acc_a, acc_b, as1, ad1, h1, b1r, W2p, a2s, a2d)


def _tc_final_body(acca, accb, asc, adc, h2, b2r, out_ref):
  e = asc[...] + adc[...]
  base = jnp.exp(jnp.where(e >= 0.0, e, 0.2 * e))
  num = acca[...] + accb[...] + base * h2[...]
  den = (acca[:, D_OUT:D_OUT + 1] + accb[:, D_OUT:D_OUT + 1] + base + 1e-16)
  z = num / den + b2r[...]
  out_ref[...] = 1.0 / (1.0 + jnp.exp(-z))


def _tc_final(acc_a, acc_b, as2, ad2, h2, b2r):
  R = 1024
  return pl.pallas_call(
      _tc_final_body,
      grid=(NP // R,),
      in_specs=[
          pl.BlockSpec((R, D), lambda i: (i, 0)),
          pl.BlockSpec((R, D), lambda i: (i, 0)),
          pl.BlockSpec((R, 1), lambda i: (i, 0)),
          pl.BlockSpec((R, 1), lambda i: (i, 0)),
          pl.BlockSpec((R, D), lambda i: (i, 0)),
          pl.BlockSpec((1, D), lambda i: (0, 0)),
      ],
      out_specs=pl.BlockSpec((R, D), lambda i: (i, 0)),
      out_shape=jax.ShapeDtypeStruct((NP, D), jnp.float32),
  )(acc_a, acc_b, as2, ad2, h2, b2r)


def kernel(x, edge_index, W1, a_src1, a_dst1, b1, W2, a_src2, a_dst2, b2):
  x_pad = jnp.zeros((NP, D_IN), jnp.float32).at[:N].set(x)
  npad = NTILES * EPT - E
  pad_idx = N + (jnp.arange(npad, dtype=jnp.int32) % (NP - N))
  srcp = jnp.concatenate([edge_index[0].astype(jnp.int32), pad_idx])
  dstp = jnp.concatenate([edge_index[1].astype(jnp.int32), pad_idx])
  edg = jnp.stack([srcp.reshape(NTILES, NCH, CH),
                   dstp.reshape(NTILES, NCH, CH)], axis=2)

  W1p = jnp.zeros((D_IN, D), jnp.float32).at[:, :D_HID].set(W1)
  a1sp = jnp.zeros((1, D), jnp.float32).at[0, :D_HID].set(a_src1)
  a1dp = jnp.zeros((1, D), jnp.float32).at[0, :D_HID].set(a_dst1)
  b1r = b1.reshape(1, D_HID)
  W2p = jnp.zeros((D_HID, D), jnp.float32).at[:, :D_OUT].set(W2)
  a2sp = jnp.zeros((1, D), jnp.float32).at[0, :D_OUT].set(a_src2)
  a2dp = jnp.zeros((1, D), jnp.float32).at[0, :D_OUT].set(a_dst2)
  b2r = jnp.zeros((1, D), jnp.float32).at[0, :D_OUT].set(b2)

  h1, as1, ad1 = _tc_front(x_pad, W1p, a1sp, a1dp)
  acc1 = _sc_agg1(edg, as1.reshape(NP), ad1.reshape(NP), h1)
  h2, as2, ad2 = _tc_mid(acc1[0], acc1[1], as1, ad1, h1, b1r, W2p,
                         a2sp, a2dp)
  acc2 = _sc_agg2(edg, as2.reshape(NP), ad2.reshape(NP), h2)
  outp = _tc_final(acc2[0], acc2[1], as2, ad2, h2, b2r)
  return outp[:N, :D_OUT]


# speedup vs baseline: 1.1032x; 1.0072x over previous
import dataclasses
import functools

import jax
import jax.numpy as jnp
from jax import lax
from jax.experimental import pallas as pl
from jax.experimental.pallas import tpu as pltpu
from jax.experimental.pallas import tpu_sc as plsc

N = 10000
NP = 10112
E = 320000
NTILES = 32
CH = 96
NCH = 106
EPT = NCH * CH
RPT = NP // 16

D = 128
D_IN = 128
D_HID = 64
D_OUT = 121

_SC_PARAMS = pltpu.CompilerParams()
if "needs_layout_passes" in pltpu.CompilerParams.__dataclass_fields__:
  _SC_PARAMS = dataclasses.replace(_SC_PARAMS, needs_layout_passes=False)


def _make_sc_agg(den_col):
  dblk, dlane = den_col // 16, den_col % 16
  mesh = plsc.VectorSubcoreMesh(core_axis_name="c", subcore_axis_name="s")

  @functools.partial(
      pl.kernel,
      compiler_params=_SC_PARAMS,
      out_type=jax.ShapeDtypeStruct((2, NP, D), jnp.float32),
      mesh=mesh,
      scratch_types=[
          pltpu.VMEM((NP,), jnp.float32),
          pltpu.VMEM((NP,), jnp.float32),
          pltpu.VMEM((2, CH), jnp.int32),
          pltpu.VMEM((2, CH), jnp.int32),
          pltpu.VMEM((1, CH), jnp.int32),
          pltpu.VMEM((1, CH), jnp.int32),
          pltpu.VMEM((CH,), jnp.float32),
          pltpu.VMEM((CH,), jnp.float32),
          pltpu.VMEM((CH, D), jnp.float32),
          pltpu.VMEM((CH, D), jnp.float32),
          pltpu.VMEM_SHARED((NP, D), jnp.float32),
          pltpu.SemaphoreType.DMA,
          pltpu.SemaphoreType.DMA,
          pltpu.SemaphoreType.DMA,
          pltpu.SemaphoreType.DMA,
          pltpu.SemaphoreType.DMA,
          pltpu.SemaphoreType.DMA,
      ],
  )
  def sc_agg(edg_hbm, asv_hbm, adv_hbm, h_hbm, out_hbm,
             as_v, ad_v, ebA, ebB, dbA, dbB, exbA, exbB, gbA, gbB, acc,
             isemA, isemB, gsemA, gsemB, ssemA, ssemB):
    cid = lax.axis_index("c")
    sid = lax.axis_index("s")
    wid = cid * 16 + sid

    pltpu.sync_copy(asv_hbm, as_v)
    pltpu.sync_copy(adv_hbm, ad_v)

    zero16 = jnp.zeros((16,), jnp.float32)
    dmask = jnp.where(lax.iota(jnp.int32, 16) == dlane, 1.0, 0.0)

    @pl.loop(0, CH)
    def _zrow(j):
      for t in range(D // 16):
        gbA[j, pl.ds(t * 16, 16)] = zero16

    for m in range(RPT // CH):
      pltpu.sync_copy(gbA, acc.at[pl.ds(sid * RPT + m * CH, CH)])
    rem = RPT - (RPT // CH) * CH
    if rem:
      pltpu.sync_copy(gbA.at[pl.ds(0, rem)],
                      acc.at[pl.ds(sid * RPT + (RPT // CH) * CH, rem)])
    plsc.subcore_barrier()

    def do_chunk(c, eb, db, exb, gb, isem, gsem, ssem,
                 ebn, dbn, gbn, isemn, gsemn, ssemn):
      for k in range(0, CH, 16):
        si = eb[0, pl.ds(k, 16)]
        di = eb[1, pl.ds(k, 16)]
        db[0, pl.ds(k, 16)] = di
        e = plsc.load_gather(as_v, [si]) + plsc.load_gather(ad_v, [di])
        e = jnp.where(e >= 0.0, e, 0.2 * e)
        exb[pl.ds(k, 16)] = jnp.exp(e)

      pltpu.make_async_copy(h_hbm.at[eb.at[0]], gb, gsem).wait()

      @pl.when(c + 2 < NCH)
      def _():
        pltpu.async_copy(edg_hbm.at[wid, c + 2], eb, isem)

      @pl.when(c + 1 < NCH)
      def _():
        pltpu.make_async_copy(edg_hbm.at[wid, c + 1], ebn, isemn).wait()

        @pl.when(c >= 1)
        def _():
          pltpu.make_async_copy(gbn, acc.at[dbn.at[0]], ssemn).wait()

        pltpu.async_copy(h_hbm.at[ebn.at[0]], gbn, gsemn)

      @pl.loop(0, CH, step=2)
      def _scale(j):
        ex0 = plsc.load_gather(exb, [jnp.zeros((16,), jnp.int32) + j])
        ex1 = plsc.load_gather(exb, [jnp.zeros((16,), jnp.int32) + (j + 1)])
        for t in range(dblk + 1):
          p0 = gb[j, pl.ds(t * 16, 16)] * ex0
          p1 = gb[j + 1, pl.ds(t * 16, 16)] * ex1
          if t == dblk:
            p0 = p0 + ex0 * dmask
            p1 = p1 + ex1 * dmask
          gb[j, pl.ds(t * 16, 16)] = p0
          gb[j + 1, pl.ds(t * 16, 16)] = p1

      pltpu.async_copy(gb, acc.at[db.at[0]], ssem, add=True)

    pltpu.async_copy(edg_hbm.at[wid, 0], ebA, isemA)
    pltpu.async_copy(edg_hbm.at[wid, 1], ebB, isemB)
    pltpu.make_async_copy(edg_hbm.at[wid, 0], ebA, isemA).wait()
    pltpu.async_copy(h_hbm.at[ebA.at[0]], gbA, gsemA)

    @pl.loop(0, NCH, step=2)
    def _chunks(c):
      do_chunk(c, ebA, dbA, exbA, gbA, isemA, gsemA, ssemA,
               ebB, dbB, gbB, isemB, gsemB, ssemB)
      do_chunk(c + 1, ebB, dbB, exbB, gbB, isemB, gsemB, ssemB,
               ebA, dbA, gbA, isemA, gsemA, ssemA)

    pltpu.make_async_copy(gbA, acc.at[dbA.at[0]], ssemA).wait()
    pltpu.make_async_copy(gbB, acc.at[dbB.at[0]], ssemB).wait()

    plsc.subcore_barrier()
    pltpu.sync_copy(acc.at[pl.ds(sid * RPT, RPT)],
                    out_hbm.at[cid, pl.ds(sid * RPT, RPT)])

  return sc_agg


_sc_agg1 = _make_sc_agg(D_HID)
_sc_agg2 = _make_sc_agg(D_OUT)


def _tc_front_body(x_ref, w_ref, asr, adr, h_ref, asc, adc):
  h = jnp.dot(x_ref[...], w_ref[...], preferred_element_type=jnp.float32,
              precision=lax.Precision.HIGHEST)
  h_ref[...] = h
  asc[...] = jnp.sum(h * asr[...], axis=1, keepdims=True)
  adc[...] = jnp.sum(h * adr[...], axis=1, keepdims=True)


def _tc_front(x_pad, W1p, a1s, a1d):
  R = 1264
  return pl.pallas_call(
      _tc_front_body,
      grid=(NP // R,),
      in_specs=[
          pl.BlockSpec((R, D_IN), lambda i: (i, 0)),
          pl.BlockSpec((D_IN, D), lambda i: (0, 0)),
          pl.BlockSpec((1, D), lambda i: (0, 0)),
          pl.BlockSpec((1, D), lambda i: (0, 0)),
      ],
      out_specs=[
          pl.BlockSpec((R, D), lambda i: (i, 0)),
          pl.BlockSpec((R, 1), lambda i: (i, 0)),
          pl.BlockSpec((R, 1), lambda i: (i, 0)),
      ],
      out_shape=[
          jax.ShapeDtypeStruct((NP, D), jnp.float32),
          jax.ShapeDtypeStruct((NP, 1), jnp.float32),
          jax.ShapeDtypeStruct((NP, 1), jnp.float32),
      ],
  )(x_pad, W1p, a1s, a1d)


def _tc_mid_body(acca, accb, asc, adc, h1, b1r, w2, a2s, a2d,
                 h2_ref, as2, ad2):
  e = asc[...] + adc[...]
  base = jnp.exp(jnp.where(e >= 0.0, e, 0.2 * e))
  num = acca[:, :D_HID] + accb[:, :D_HID] + base * h1[:, :D_HID]
  den = (acca[:, D_HID:D_HID + 1] + accb[:, D_HID:D_HID + 1] + base + 1e-16)
  g = jnp.maximum(num / den + b1r[...], 0.0)
  h2 = jnp.dot(g, w2[...], preferred_element_type=jnp.float32,
               precision=lax.Precision.HIGHEST)
  h2_ref[...] = h2
  as2[...] = jnp.sum(h2 * a2s[...], axis=1, keepdims=True)
  ad2[...] = jnp.sum(h2 * a2d[...], axis=1, keepdims=True)


def _tc_mid(acc_a, acc_b, as1, ad1, h1, b1r, W2p, a2s, a2d):
  R = 1264
  return pl.pallas_call(
      _tc_mid_body,
      grid=(NP // R,),
      in_specs=[
          pl.BlockSpec((R, D), lambda i: (i, 0)),
          pl.BlockSpec((R, D), lambda i: (i, 0)),
          pl.BlockSpec((R, 1), lambda i: (i, 0)),
          pl.BlockSpec((R, 1), lambda i: (i, 0)),
          pl.BlockSpec((R, D), lambda i: (i, 0)),
          pl.BlockSpec((1, D_HID), lambda i: (0, 0)),
          pl.BlockSpec((D_HID, D), lambda i: (0, 0)),
          pl.BlockSpec((1, D), lambda i: (0, 0)),
          pl.BlockSpec((1, D), lambda i: (0, 0)),
      ],
      out_specs=[
          pl.BlockSpec((R, D), lambda i: (i, 0)),
          pl.BlockSpec((R, 1), lambda i: (i, 0)),
          pl.BlockSpec((R, 1), lambda i: (i, 0)),
      ],
      out_shape=[
          jax.ShapeDtypeStruct((NP, D), jnp.float32),
          jax.ShapeDtypeStruct((NP, 1), jnp.float32),
          jax.ShapeDtypeStruct((NP, 1), jnp.float32),
      ],
  )(acc_a, acc_b, as1, ad1, h1, b1r, W2p, a2s, a2d)


def _tc_final_body(acca, accb, asc, adc, h2, b2r, out_ref):
  e = asc[...] + adc[...]
  base = jnp.exp(jnp.where(e >= 0.0, e, 0.2 * e))
  num = acca[...] + accb[...] + base * h2[...]
  den = (acca[:, D_OUT:D_OUT + 1] + accb[:, D_OUT:D_OUT + 1] + base + 1e-16)
  z = num / den + b2r[...]
  out_ref[...] = 1.0 / (1.0 + jnp.exp(-z))


def _tc_final(acc_a, acc_b, as2, ad2, h2, b2r):
  R = 1264
  return pl.pallas_call(
      _tc_final_body,
      grid=(NP // R,),
      in_specs=[
          pl.BlockSpec((R, D), lambda i: (i, 0)),
          pl.BlockSpec((R, D), lambda i: (i, 0)),
          pl.BlockSpec((R, 1), lambda i: (i, 0)),
          pl.BlockSpec((R, 1), lambda i: (i, 0)),
          pl.BlockSpec((R, D), lambda i: (i, 0)),
          pl.BlockSpec((1, D), lambda i: (0, 0)),
      ],
      out_specs=pl.BlockSpec((R, D), lambda i: (i, 0)),
      out_shape=jax.ShapeDtypeStruct((NP, D), jnp.float32),
  )(acc_a, acc_b, as2, ad2, h2, b2r)


def kernel(x, edge_index, W1, a_src1, a_dst1, b1, W2, a_src2, a_dst2, b2):
  x_pad = x
  npad = NTILES * EPT - E
  pad_idx = N + (jnp.arange(npad, dtype=jnp.int32) % (NP - N))
  srcp = jnp.concatenate([edge_index[0].astype(jnp.int32), pad_idx])
  dstp = jnp.concatenate([edge_index[1].astype(jnp.int32), pad_idx])
  edg = jnp.stack([srcp.reshape(NTILES, NCH, CH),
                   dstp.reshape(NTILES, NCH, CH)], axis=2)

  W1p = jnp.zeros((D_IN, D), jnp.float32).at[:, :D_HID].set(W1)
  a1sp = jnp.zeros((1, D), jnp.float32).at[0, :D_HID].set(a_src1)
  a1dp = jnp.zeros((1, D), jnp.float32).at[0, :D_HID].set(a_dst1)
  b1r = b1.reshape(1, D_HID)
  W2p = jnp.zeros((D_HID, D), jnp.float32).at[:, :D_OUT].set(W2)
  a2sp = jnp.zeros((1, D), jnp.float32).at[0, :D_OUT].set(a_src2)
  a2dp = jnp.zeros((1, D), jnp.float32).at[0, :D_OUT].set(a_dst2)
  b2r = jnp.zeros((1, D), jnp.float32).at[0, :D_OUT].set(b2)

  h1, as1, ad1 = _tc_front(x_pad, W1p, a1sp, a1dp)
  acc1 = _sc_agg1(edg, as1.reshape(NP), ad1.reshape(NP), h1)
  h2, as2, ad2 = _tc_mid(acc1[0], acc1[1], as1, ad1, h1, b1r, W2p,
                         a2sp, a2dp)
  acc2 = _sc_agg2(edg, as2.reshape(NP), ad2.reshape(NP), h2)
  outp = _tc_final(acc2[0], acc2[1], as2, ad2, h2, b2r)
  return outp[:N, :D_OUT]
